# baseline (speedup 1.0000x reference)
import functools

import jax
import jax.numpy as jnp
from jax import lax
from jax.experimental import pallas as pl
from jax.experimental.pallas import tpu as pltpu
from jax.experimental.pallas import tpu_sc as plsc

_NC, _NS = 2, 16
_NW = _NC * _NS
_G = 128
_CB = 2
_L = 16


def kernel(input_ids, word_table, pos_table):
    B, S = input_ids.shape
    V, D = word_table.shape
    N = B * S
    NBLK = N // _G
    PBLK = S // _G
    R = _CB * _G
    assert _NW * R == S
    niter = B

    pos_r = pos_table.reshape(PBLK, _G, D)

    mesh = plsc.VectorSubcoreMesh(core_axis_name="c", subcore_axis_name="s")

    scratch = (
        [pltpu.VMEM((R,), jnp.int32) for _ in range(niter)]
        + [pltpu.VMEM((_CB, _G, D), jnp.float32) for _ in range(2)]
        + [pltpu.VMEM((_CB, _G, D), jnp.float32)]
        + [pltpu.SemaphoreType.DMA for _ in range(niter + 2 + 2 + 1 + 2)]
    )
    prefill_mode = tuple(i % 2 == 0 for i in range(niter))

    @functools.partial(
        pl.kernel,
        out_type=jax.ShapeDtypeStruct((B, S, D), jnp.float32),
        mesh=mesh,
        scratch_types=scratch,
    )
    def emb(ids_hbm, wt_hbm, pt_hbm, out_hbm, *sc):
        idx_bufs = sc[0:niter]
        row_bufs = sc[niter:niter + 2]
        pos_buf = sc[niter + 2]
        lsems = sc[niter + 3:2 * niter + 3]
        gsems = sc[2 * niter + 3:2 * niter + 5]
        osems = sc[2 * niter + 5:2 * niter + 7]
        psem = sc[2 * niter + 7]
        fsems = sc[2 * niter + 8:2 * niter + 10]

        wid = lax.axis_index("s") * _NC + lax.axis_index("c")
        pblk0 = wid * _CB

        idxd = [
            pltpu.async_copy(ids_hbm.at[c, pl.ds(pblk0 * _G, R)],
                             idx_bufs[c], lsems[c])
            for c in range(niter)
        ]
        posd = pltpu.async_copy(pt_hbm.at[pl.ds(pblk0, _CB)], pos_buf, psem)

        def start_prefill(i):
            b = i % 2
            return pltpu.async_copy(pt_hbm.at[pl.ds(pblk0, _CB)],
                                    row_bufs[b], fsems[b])

        def start_gathers(i):
            b = i % 2
            return [
                pltpu.async_copy(wt_hbm.at[idx_bufs[i].at[pl.ds(k * _G, _G)]],
                                 row_bufs[b].at[k], gsems[b],
                                 add=prefill_mode[i])
                for k in range(_CB)
            ]

        def add_pos_block(i, blkk):
            b = i % 2

            @plsc.parallel_loop(0, _G, unroll=4)
            def body(r):
                for c0 in range(0, D, _L):
                    v = pos_buf[blkk, r, pl.ds(c0, _L)]
                    plsc.addupdate(
                        row_bufs[b].at[blkk, r, pl.ds(c0, _L)], v)

        gath, stores = {}, {}
        idxd[0].wait()
        if prefill_mode[0]:
            pre0 = start_prefill(0)
            pre0.wait()
        gath[0] = start_gathers(0)

        for i in range(niter):
            b = i % 2
            if i + 1 < niter:
                idxd[i + 1].wait()
                if i >= 1:
                    for d in stores[i - 1]:
                        d.wait()
                if prefill_mode[i + 1]:
                    pre = start_prefill(i + 1)
                    pre.wait()
                gath[i + 1] = start_gathers(i + 1)
            if i == 0:
                posd.wait()
            blk_stores = []
            for k in range(_CB):
                gath[i][k].wait()
                if not prefill_mode[i]:
                    add_pos_block(i, k)
                blk_stores.append(pltpu.async_copy(
                    row_bufs[b].at[k],
                    out_hbm.at[i, pl.ds((pblk0 + k) * _G, _G)], osems[b]))
            stores[i] = blk_stores

        for i in range(max(0, niter - 2), niter):
            for d in stores[i]:
                d.wait()

    return emb(input_ids, word_table, pos_r)


# speedup vs baseline: 1.0274x; 1.0274x over previous
import functools

import jax
import jax.numpy as jnp
from jax import lax
from jax.experimental import pallas as pl
from jax.experimental.pallas import tpu as pltpu
from jax.experimental.pallas import tpu_sc as plsc

_NC, _NS = 2, 16
_NW = _NC * _NS
_G = 64
_KB = 4
_NB = 8
_LEAD = 6
_L = 16


def kernel(input_ids, word_table, pos_table):
    B, S = input_ids.shape
    V, D = word_table.shape
    R = _KB * _G
    assert _NW * R == S
    nblk = B * _KB

    pos_r = pos_table.reshape(S // _G, _G, D)

    mesh = plsc.VectorSubcoreMesh(core_axis_name="c", subcore_axis_name="s")

    scratch = (
        [pltpu.VMEM((R,), jnp.int32) for _ in range(B)]
        + [pltpu.VMEM((_G, D), jnp.float32) for _ in range(_NB)]
        + [pltpu.VMEM((_KB, _G, D), jnp.float32)]
        + [pltpu.SemaphoreType.DMA for _ in range(B + 2 * _NB + 1)]
    )

    @functools.partial(
        pl.kernel,
        out_type=jax.ShapeDtypeStruct((B, S, D), jnp.float32),
        mesh=mesh,
        scratch_types=scratch,
    )
    def emb(ids_hbm, wt_hbm, pt_hbm, out_hbm, *sc):
        idx_bufs = sc[0:B]
        row_bufs = sc[B:B + _NB]
        pos_buf = sc[B + _NB]
        lsems = sc[B + _NB + 1:2 * B + _NB + 1]
        gsems = sc[2 * B + _NB + 1:2 * B + 2 * _NB + 1]
        osems = sc[2 * B + 2 * _NB + 1:2 * B + 3 * _NB + 1]
        psem = sc[2 * B + 3 * _NB + 1]

        wid = lax.axis_index("s") * _NC + lax.axis_index("c")
        row0 = wid * R

        idxd = [
            pltpu.async_copy(ids_hbm.at[b, pl.ds(row0, R)], idx_bufs[b],
                             lsems[b])
            for b in range(B)
        ]
        posd = pltpu.async_copy(pt_hbm.at[pl.ds(wid * _KB, _KB)], pos_buf,
                                psem)

        def start_gather(j):
            m = j % _NB
            b, k = j // _KB, j % _KB
            return pltpu.async_copy(
                wt_hbm.at[idx_bufs[b].at[pl.ds(k * _G, _G)]],
                row_bufs[m], gsems[m])

        def add_pos(j):
            m = j % _NB
            k = j % _KB

            @plsc.parallel_loop(0, _G, unroll=4)
            def body(r):
                for c0 in range(0, D, _L):
                    v = pos_buf[k, r, pl.ds(c0, _L)]
                    plsc.addupdate(row_bufs[m].at[r, pl.ds(c0, _L)], v)

        for d in idxd:
            d.wait()
        gath, stores = {}, {}
        for j in range(min(_LEAD, nblk)):
            gath[j] = start_gather(j)
        posd.wait()

        for j in range(nblk):
            m = j % _NB
            b, k = j // _KB, j % _KB
            gath[j].wait()
            add_pos(j)
            stores[j] = pltpu.async_copy(
                row_bufs[m],
                out_hbm.at[b, pl.ds(row0 + k * _G, _G)], osems[m])
            g = j + _LEAD
            if g < nblk:
                if g >= _NB:
                    stores[g - _NB].wait()
                gath[g] = start_gather(g)

        for j in range(max(0, nblk - _NB), nblk):
            stores[j].wait()

    return emb(input_ids, word_table, pos_r)
